# baseline (speedup 1.0000x reference)
import jax
import jax.numpy as jnp
from jax import lax
from jax.experimental import pallas as pl
from jax.experimental.pallas import tpu as pltpu
from jax.experimental.pallas import tpu_sc as plsc

B = 16384
L = 200
D = 8
NW = 32
ROWS_PER_W = B // NW
CB = 16
NCH = ROWS_PER_W // CB
CH_IDX = CB * L


def _pool_body(x_hbm, emb_hbm, out_hbm, idx_v, rows_v, pooled_v, sem):
    wid = lax.axis_index("c") * 16 + lax.axis_index("s")
    base_idx = wid * (ROWS_PER_W * L)
    inv_l = jnp.float32(1.0 / L)

    l16 = lax.iota(jnp.int32, 16)
    col = lax.bitwise_and(l16, 7)
    half = lax.shift_right_logical(l16, 3) * L

    def chunk_body(g, carry):
        pltpu.sync_copy(x_hbm.at[pl.ds(base_idx + g * CH_IDX, CH_IDX)], idx_v)
        pltpu.async_copy(emb_hbm.at[idx_v], rows_v, sem).wait()

        def jbody(j, accs):
            rj = half + j
            return tuple(
                acc + plsc.load_gather(rows_v, [(2 * p) * L + rj, col])
                for p, acc in enumerate(accs))

        accs = lax.fori_loop(
            0, L, jbody,
            tuple(jnp.zeros((16,), jnp.float32) for _ in range(CB // 2)))
        for p, acc in enumerate(accs):
            pooled_v[pl.ds(g * (CB * D) + p * 16, 16)] = acc * inv_l
        return carry

    lax.fori_loop(0, NCH, chunk_body, 0)
    pltpu.sync_copy(pooled_v, out_hbm.at[pl.ds(wid * (ROWS_PER_W * D),
                                               ROWS_PER_W * D)])


_pool = pl.kernel(
    _pool_body,
    out_type=jax.ShapeDtypeStruct((B * D,), jnp.float32),
    mesh=plsc.VectorSubcoreMesh(core_axis_name="c", subcore_axis_name="s"),
    compiler_params=pltpu.CompilerParams(needs_layout_passes=False,
                                         use_tc_tiling_on_sc=False),
    scratch_types=[
        pltpu.VMEM((CH_IDX,), jnp.int32),
        pltpu.VMEM((CH_IDX, D), jnp.float32),
        pltpu.VMEM((ROWS_PER_W * D,), jnp.float32),
        pltpu.SemaphoreType.DMA,
    ],
)


FMT_GRID = 62
FMT_COLS = 16384
VP = FMT_GRID * FMT_COLS


def _fmt_body(in_ref, out_ref):
    t = in_ref[...].reshape(D, FMT_COLS // 16, 16)
    t = t.transpose(1, 2, 0)
    out_ref[...] = t.reshape(FMT_COLS // 16, 128)


_fmt = pl.pallas_call(
    _fmt_body,
    grid=(FMT_GRID,),
    in_specs=[pl.BlockSpec((D, FMT_COLS), lambda k: (0, k))],
    out_specs=pl.BlockSpec((FMT_COLS // 16, 128), lambda k: (k, 0)),
    out_shape=jax.ShapeDtypeStruct((VP // 16, 128), jnp.float32),
)


def _mlp_body(h_ref, w1_ref, b1_ref, w2_ref, b2_ref, out_ref):
    h = h_ref[...]
    a = jnp.dot(h, w1_ref[...], preferred_element_type=jnp.float32)
    a = jnp.maximum(a + b1_ref[...], 0.0)
    z = jnp.sum(a * w2_ref[...][:, 0][None, :], axis=1, keepdims=True)
    z = z + b2_ref[...]
    out_ref[...] = 1.0 / (1.0 + jnp.exp(-z))


def _mlp(pooled, w1, b1, w2, b2):
    return pl.pallas_call(
        _mlp_body,
        out_shape=jax.ShapeDtypeStruct((B, 1), jnp.float32),
    )(pooled, w1, b1, w2, b2)


@jax.jit
def kernel(x, emb, W1, b1, W2, b2):
    x_flat = x.reshape(-1).astype(jnp.int32)
    emb2 = _fmt(emb.T).reshape(-1).reshape(VP, D)
    pooled = _pool(x_flat, emb2).reshape(B, D)
    return _mlp(pooled, W1, b1, W2, b2)


# speedup vs baseline: 1.1194x; 1.1194x over previous
import jax
import jax.numpy as jnp
from jax import lax
from jax.experimental import pallas as pl
from jax.experimental.pallas import tpu as pltpu
from jax.experimental.pallas import tpu_sc as plsc

B = 16384
L = 200
D = 8
NW = 32
ROWS_PER_W = B // NW
CB = 16
NCH = ROWS_PER_W // CB
CH_IDX = CB * L


def _pool_body(x_hbm, emb_hbm, out_hbm, idx0, idx1, rows0, rows1, pooled_v,
               sem0, sem1):
    wid = lax.axis_index("c") * 16 + lax.axis_index("s")
    base_idx = wid * (ROWS_PER_W * L)
    inv_l = jnp.float32(1.0 / L)

    l16 = lax.iota(jnp.int32, 16)
    col = lax.bitwise_and(l16, 7)
    half = lax.shift_right_logical(l16, 3) * L

    idx_bufs = (idx0, idx1)
    row_bufs = (rows0, rows1)
    sems = (sem0, sem1)

    def start(g):
        s = g % 2
        pltpu.sync_copy(x_hbm.at[pl.ds(base_idx + g * CH_IDX, CH_IDX)],
                        idx_bufs[s])
        pltpu.async_copy(emb_hbm.at[idx_bufs[s]], row_bufs[s], sems[s])

    start(0)
    for g in range(NCH):
        s = g % 2
        if g + 1 < NCH:
            start(g + 1)
        pltpu.make_async_copy(emb_hbm.at[idx_bufs[s]], row_bufs[s],
                              sems[s]).wait()
        rows_v = row_bufs[s]

        def jbody(j, accs):
            rj = half + j
            return tuple(
                acc + plsc.load_gather(rows_v, [(2 * p) * L + rj, col])
                for p, acc in enumerate(accs))

        accs = lax.fori_loop(
            0, L, jbody,
            tuple(jnp.zeros((16,), jnp.float32) for _ in range(CB // 2)))
        for p, acc in enumerate(accs):
            pooled_v[pl.ds(g * (CB * D) + p * 16, 16)] = acc * inv_l

    pltpu.sync_copy(pooled_v, out_hbm.at[pl.ds(wid * (ROWS_PER_W * D),
                                               ROWS_PER_W * D)])


_pool = pl.kernel(
    _pool_body,
    out_type=jax.ShapeDtypeStruct((B * D,), jnp.float32),
    mesh=plsc.VectorSubcoreMesh(core_axis_name="c", subcore_axis_name="s"),
    compiler_params=pltpu.CompilerParams(needs_layout_passes=False,
                                         use_tc_tiling_on_sc=False),
    scratch_types=[
        pltpu.VMEM((CH_IDX,), jnp.int32),
        pltpu.VMEM((CH_IDX,), jnp.int32),
        pltpu.VMEM((CH_IDX, D), jnp.float32),
        pltpu.VMEM((CH_IDX, D), jnp.float32),
        pltpu.VMEM((ROWS_PER_W * D,), jnp.float32),
        pltpu.SemaphoreType.DMA,
        pltpu.SemaphoreType.DMA,
    ],
)


FMT_GRID = 62
FMT_COLS = 16384
VP = FMT_GRID * FMT_COLS


def _fmt_body(in_ref, out_ref):
    t = in_ref[...].reshape(D, FMT_COLS // 16, 16)
    t = t.transpose(1, 2, 0)
    out_ref[...] = t.reshape(FMT_COLS // 16, 128)


_fmt = pl.pallas_call(
    _fmt_body,
    grid=(FMT_GRID,),
    in_specs=[pl.BlockSpec((D, FMT_COLS), lambda k: (0, k))],
    out_specs=pl.BlockSpec((FMT_COLS // 16, 128), lambda k: (k, 0)),
    out_shape=jax.ShapeDtypeStruct((VP // 16, 128), jnp.float32),
)


def _mlp_body(h_ref, w1_ref, b1_ref, w2_ref, b2_ref, out_ref):
    h = h_ref[...]
    a = jnp.dot(h, w1_ref[...], preferred_element_type=jnp.float32)
    a = jnp.maximum(a + b1_ref[...], 0.0)
    z = jnp.sum(a * w2_ref[...][:, 0][None, :], axis=1, keepdims=True)
    z = z + b2_ref[...]
    out_ref[...] = 1.0 / (1.0 + jnp.exp(-z))


def _mlp(pooled, w1, b1, w2, b2):
    return pl.pallas_call(
        _mlp_body,
        out_shape=jax.ShapeDtypeStruct((B, 1), jnp.float32),
    )(pooled, w1, b1, w2, b2)


@jax.jit
def kernel(x, emb, W1, b1, W2, b2):
    x_flat = x.reshape(-1).astype(jnp.int32)
    emb2 = _fmt(emb.T).reshape(-1).reshape(VP, D)
    pooled = _pool(x_flat, emb2).reshape(B, D)
    return _mlp(pooled, W1, b1, W2, b2)


# speedup vs baseline: 2.7206x; 2.4303x over previous
import jax
import jax.numpy as jnp
from jax import lax
from jax.experimental import pallas as pl
from jax.experimental.pallas import tpu as pltpu
from jax.experimental.pallas import tpu_sc as plsc

B = 16384
L = 200
D = 8
NW = 32
ROWS_PER_W = B // NW
CB = 16
NCH = ROWS_PER_W // CB
CH_IDX = CB * L


def _pool_body(x_hbm, emb_hbm, out_hbm, idx0, idx1, rows0, rows1, pooled_v,
               sem0, sem1):
    wid = lax.axis_index("c") * 16 + lax.axis_index("s")
    base_idx = wid * (ROWS_PER_W * L)
    inv_l = jnp.float32(1.0 / L)

    l16 = lax.iota(jnp.int32, 16)
    col = lax.bitwise_and(l16, 7)
    half = lax.shift_right_logical(l16, 3) * L

    idx_bufs = (idx0, idx1)
    row_bufs = (rows0, rows1)
    sems = (sem0, sem1)

    def start(g):
        s = g % 2
        pltpu.sync_copy(x_hbm.at[pl.ds(base_idx + g * CH_IDX, CH_IDX)],
                        idx_bufs[s])
        pltpu.async_copy(emb_hbm.at[idx_bufs[s]], row_bufs[s], sems[s])

    start(0)
    for g in range(NCH):
        s = g % 2
        if g + 1 < NCH:
            start(g + 1)
        pltpu.make_async_copy(emb_hbm.at[idx_bufs[s]], row_bufs[s],
                              sems[s]).wait()
        rows_v = row_bufs[s]

        def jbody(j, accs):
            rj = half + j
            return tuple(
                acc + plsc.load_gather(rows_v, [(2 * p) * L + rj, col])
                for p, acc in enumerate(accs))

        accs = lax.fori_loop(
            0, L, jbody,
            tuple(jnp.zeros((16,), jnp.float32) for _ in range(CB // 2)))
        for p, acc in enumerate(accs):
            pooled_v[pl.ds(g * (CB * D) + p * 16, 16)] = acc * inv_l

    pltpu.sync_copy(pooled_v, out_hbm.at[pl.ds(wid * (ROWS_PER_W * D),
                                               ROWS_PER_W * D)])


_pool = pl.kernel(
    _pool_body,
    out_type=jax.ShapeDtypeStruct((B * D,), jnp.float32),
    mesh=plsc.VectorSubcoreMesh(core_axis_name="c", subcore_axis_name="s"),
    compiler_params=pltpu.CompilerParams(needs_layout_passes=False,
                                         use_tc_tiling_on_sc=False),
    scratch_types=[
        pltpu.VMEM((CH_IDX,), jnp.int32),
        pltpu.VMEM((CH_IDX,), jnp.int32),
        pltpu.VMEM((CH_IDX, D), jnp.float32),
        pltpu.VMEM((CH_IDX, D), jnp.float32),
        pltpu.VMEM((ROWS_PER_W * D,), jnp.float32),
        pltpu.SemaphoreType.DMA,
        pltpu.SemaphoreType.DMA,
    ],
)


FMT_GRID = 62
FMT_COLS = 16384
VP = FMT_GRID * FMT_COLS


def _fmt_body(in_ref, out_ref):
    x = in_ref[...]
    for g in range(FMT_COLS // 2048):
        r5 = jnp.concatenate(
            [x[:, g * 2048 + 128 * t: g * 2048 + 128 * (t + 1)]
             for t in range(16)], axis=0)
        out_ref[pl.ds(g * 128, 128), :] = r5.T


_fmt = pl.pallas_call(
    _fmt_body,
    grid=(FMT_GRID,),
    in_specs=[pl.BlockSpec((D, FMT_COLS), lambda k: (0, k))],
    out_specs=pl.BlockSpec((FMT_COLS // 16, 128), lambda k: (k, 0)),
    out_shape=jax.ShapeDtypeStruct((VP // 16, 128), jnp.float32),
)


def _mlp_body(h_ref, w1_ref, b1_ref, w2_ref, b2_ref, out_ref):
    h = h_ref[...]
    a = jnp.dot(h, w1_ref[...], preferred_element_type=jnp.float32)
    a = jnp.maximum(a + b1_ref[...], 0.0)
    z = jnp.sum(a * w2_ref[...][:, 0][None, :], axis=1, keepdims=True)
    z = z + b2_ref[...]
    out_ref[...] = 1.0 / (1.0 + jnp.exp(-z))


def _mlp(pooled, w1, b1, w2, b2):
    return pl.pallas_call(
        _mlp_body,
        out_shape=jax.ShapeDtypeStruct((B, 1), jnp.float32),
    )(pooled, w1, b1, w2, b2)


@jax.jit
def kernel(x, emb, W1, b1, W2, b2):
    xi = x.astype(jnp.int32)
    xm = ((xi & ~jnp.int32(2047)) | ((xi & 127) << 4) | ((xi >> 7) & 15))
    x_flat = xm.reshape(-1)
    emb2 = _fmt(emb.T).reshape(-1).reshape(VP, D)
    pooled = _pool(x_flat, emb2).reshape(B, D)
    return _mlp(pooled, W1, b1, W2, b2)


# speedup vs baseline: 2.7756x; 1.0202x over previous
import jax
import jax.numpy as jnp
from jax import lax
from jax.experimental import pallas as pl
from jax.experimental.pallas import tpu as pltpu
from jax.experimental.pallas import tpu_sc as plsc

B = 16384
L = 200
D = 8
NW = 32
ROWS_PER_W = B // NW
CB = 16
NCH = ROWS_PER_W // CB
CH_IDX = CB * L


NBUF = 3


def _pool_body(x_hbm, emb_hbm, out_hbm, idx0, idx1, idx2, rows0, rows1,
               rows2, pooled_v, sem0, sem1, sem2):
    wid = lax.axis_index("c") * 16 + lax.axis_index("s")
    base_idx = wid * (ROWS_PER_W * L)
    inv_l = jnp.float32(1.0 / L)

    l16 = lax.iota(jnp.int32, 16)
    col = lax.bitwise_and(l16, 7)
    half = lax.shift_right_logical(l16, 3) * L

    idx_bufs = (idx0, idx1, idx2)
    row_bufs = (rows0, rows1, rows2)
    sems = (sem0, sem1, sem2)

    def start(g):
        s = g % NBUF
        pltpu.sync_copy(x_hbm.at[pl.ds(base_idx + g * CH_IDX, CH_IDX)],
                        idx_bufs[s])
        pltpu.async_copy(emb_hbm.at[idx_bufs[s]], row_bufs[s], sems[s])

    for g in range(NBUF - 1):
        start(g)
    for g in range(NCH):
        s = g % NBUF
        if g + NBUF - 1 < NCH:
            start(g + NBUF - 1)
        pltpu.make_async_copy(emb_hbm.at[idx_bufs[s]], row_bufs[s],
                              sems[s]).wait()
        rows_v = row_bufs[s]

        def jbody(j, accs):
            out = accs
            for u in range(2):
                rj = half + (2 * j + u)
                out = tuple(
                    acc + plsc.load_gather(rows_v, [(2 * p) * L + rj, col])
                    for p, acc in enumerate(out))
            return out

        accs = lax.fori_loop(
            0, L // 2, jbody,
            tuple(jnp.zeros((16,), jnp.float32) for _ in range(CB // 2)))
        for p, acc in enumerate(accs):
            pooled_v[pl.ds(g * (CB * D) + p * 16, 16)] = acc * inv_l

    pltpu.sync_copy(pooled_v, out_hbm.at[pl.ds(wid * (ROWS_PER_W * D),
                                               ROWS_PER_W * D)])


_pool = pl.kernel(
    _pool_body,
    out_type=jax.ShapeDtypeStruct((B * D,), jnp.float32),
    mesh=plsc.VectorSubcoreMesh(core_axis_name="c", subcore_axis_name="s"),
    compiler_params=pltpu.CompilerParams(needs_layout_passes=False,
                                         use_tc_tiling_on_sc=False),
    scratch_types=[
        pltpu.VMEM((CH_IDX,), jnp.int32),
        pltpu.VMEM((CH_IDX,), jnp.int32),
        pltpu.VMEM((CH_IDX,), jnp.int32),
        pltpu.VMEM((CH_IDX, D), jnp.float32),
        pltpu.VMEM((CH_IDX, D), jnp.float32),
        pltpu.VMEM((CH_IDX, D), jnp.float32),
        pltpu.VMEM((ROWS_PER_W * D,), jnp.float32),
        pltpu.SemaphoreType.DMA,
        pltpu.SemaphoreType.DMA,
        pltpu.SemaphoreType.DMA,
    ],
)


FMT_GRID = 62
FMT_COLS = 16384
VP = FMT_GRID * FMT_COLS


def _fmt_body(in_ref, out_ref):
    x = in_ref[...]
    for g in range(FMT_COLS // 2048):
        r5 = jnp.concatenate(
            [x[:, g * 2048 + 128 * t: g * 2048 + 128 * (t + 1)]
             for t in range(16)], axis=0)
        out_ref[pl.ds(g * 128, 128), :] = r5.T


_fmt = pl.pallas_call(
    _fmt_body,
    grid=(FMT_GRID,),
    in_specs=[pl.BlockSpec((D, FMT_COLS), lambda k: (0, k))],
    out_specs=pl.BlockSpec((FMT_COLS // 16, 128), lambda k: (k, 0)),
    out_shape=jax.ShapeDtypeStruct((VP // 16, 128), jnp.float32),
)


def _mlp_body(h_ref, w1_ref, b1_ref, w2_ref, b2_ref, out_ref):
    h = h_ref[...]
    a = jnp.dot(h, w1_ref[...], preferred_element_type=jnp.float32)
    a = jnp.maximum(a + b1_ref[...], 0.0)
    z = jnp.sum(a * w2_ref[...][:, 0][None, :], axis=1, keepdims=True)
    z = z + b2_ref[...]
    out_ref[...] = 1.0 / (1.0 + jnp.exp(-z))


def _mlp(pooled, w1, b1, w2, b2):
    return pl.pallas_call(
        _mlp_body,
        out_shape=jax.ShapeDtypeStruct((B, 1), jnp.float32),
    )(pooled, w1, b1, w2, b2)


@jax.jit
def kernel(x, emb, W1, b1, W2, b2):
    xi = x.astype(jnp.int32)
    xm = ((xi & ~jnp.int32(2047)) | ((xi & 127) << 4) | ((xi >> 7) & 15))
    x_flat = xm.reshape(-1)
    emb2 = _fmt(emb.T).reshape(-1).reshape(VP, D)
    pooled = _pool(x_flat, emb2).reshape(B, D)
    return _mlp(pooled, W1, b1, W2, b2)
